# Initial kernel scaffold; baseline (speedup 1.0000x reference)
import jax
import jax.numpy as jnp
from jax.experimental import pallas as pl


def kernel(x, edge_index, W1, b1, W2, b2):
    raise NotImplementedError("write your pallas kernel here")


# speedup vs baseline: 7.9400x; 7.9400x over previous
import functools

import jax
import jax.numpy as jnp
from jax import lax
from jax.experimental import pallas as pl
from jax.experimental.pallas import tpu as pltpu
from jax.experimental.pallas import tpu_sc as plsc

N = 10000
NP = 10240
E = 320000
D = 128
NC = 2
NS = 16
NW = NC * NS
EP = 10240
CH = 128
NCH = EP // CH
RPS = NP // NS
DEGW = 128

_MESH = plsc.VectorSubcoreMesh(core_axis_name="c", subcore_axis_name="s",
                               num_cores=NC, num_subcores=NS)


@functools.partial(
    pl.kernel,
    out_type=jax.ShapeDtypeStruct((NC, NP, DEGW), jnp.float32),
    mesh=_MESH,
    scratch_types=[
        pltpu.VMEM((NCH, CH), jnp.int32),
        pltpu.VMEM((CH, DEGW), jnp.float32),
        pltpu.VMEM_SHARED((NP, DEGW), jnp.float32),
    ],
)
def _sc_degree(dst_hbm, zero_hbm, ones_hbm, out_hbm, idx_v, ones_v, acc):
    cid = lax.axis_index("c")
    sid = lax.axis_index("s")
    wid = sid * NC + cid
    pltpu.sync_copy(zero_hbm.at[pl.ds(0, RPS)], acc.at[pl.ds(sid * RPS, RPS)])
    pltpu.sync_copy(ones_hbm, ones_v)
    pltpu.sync_copy(dst_hbm.at[wid], idx_v)
    plsc.subcore_barrier()

    def body(j, carry):
        pltpu.sync_copy(ones_v, acc.at[idx_v.at[j]], add=True)
        return carry

    lax.fori_loop(0, NCH, body, 0)
    plsc.subcore_barrier()
    pltpu.sync_copy(acc.at[pl.ds(sid * RPS, RPS)],
                    out_hbm.at[cid, pl.ds(sid * RPS, RPS)])


@functools.partial(
    pl.kernel,
    out_type=jax.ShapeDtypeStruct((NC, NP, D), jnp.float32),
    mesh=_MESH,
    scratch_types=[
        pltpu.VMEM((NCH, CH), jnp.int32),
        pltpu.VMEM((NCH, CH), jnp.int32),
        pltpu.VMEM((CH, D), jnp.float32),
        pltpu.VMEM_SHARED((NP, D), jnp.float32),
        pltpu.SemaphoreType.DMA,
    ],
)
def _sc_scatter(table_hbm, src_hbm, dst_hbm, zero_hbm, out_hbm,
                idx_s, idx_d, rows_v, acc, sem):
    cid = lax.axis_index("c")
    sid = lax.axis_index("s")
    wid = sid * NC + cid
    pltpu.sync_copy(zero_hbm, acc.at[pl.ds(sid * RPS, RPS)])
    pltpu.sync_copy(src_hbm.at[wid], idx_s)
    pltpu.sync_copy(dst_hbm.at[wid], idx_d)
    plsc.subcore_barrier()

    def body(j, carry):
        pltpu.async_copy(table_hbm.at[idx_s.at[j]], rows_v, sem).wait()
        pltpu.sync_copy(rows_v, acc.at[idx_d.at[j]], add=True)
        return carry

    lax.fori_loop(0, NCH, body, 0)
    plsc.subcore_barrier()
    pltpu.sync_copy(acc.at[pl.ds(sid * RPS, RPS)],
                    out_hbm.at[cid, pl.ds(sid * RPS, RPS)])


_BLK = 1280


def _prep_body(x_ref, w_ref, d0_ref, d1_ref, t_ref, dinv_ref):
    dp = d0_ref[...] + d1_ref[...]
    deg = 1.0 + jnp.sum(dp, axis=1, keepdims=True) * (1.0 / DEGW)
    dinv = lax.rsqrt(deg)
    xw = jnp.dot(x_ref[...], w_ref[...], preferred_element_type=jnp.float32)
    t_ref[...] = dinv * xw
    dinv_ref[...] = dinv


def _tc_prep(x, w1, d0, d1):
    return pl.pallas_call(
        _prep_body,
        grid=(NP // _BLK,),
        in_specs=[
            pl.BlockSpec((_BLK, D), lambda i: (i, 0)),
            pl.BlockSpec((D, D), lambda i: (0, 0)),
            pl.BlockSpec((_BLK, DEGW), lambda i: (i, 0)),
            pl.BlockSpec((_BLK, DEGW), lambda i: (i, 0)),
        ],
        out_specs=[
            pl.BlockSpec((_BLK, D), lambda i: (i, 0)),
            pl.BlockSpec((_BLK, 1), lambda i: (i, 0)),
        ],
        out_shape=[
            jax.ShapeDtypeStruct((NP, D), jnp.float32),
            jax.ShapeDtypeStruct((NP, 1), jnp.float32),
        ],
    )(x, w1, d0, d1)


def _mid_body(p0_ref, p1_ref, t_ref, dinv_ref, b_ref, w_ref, out_ref):
    dinv = dinv_ref[...]
    h = dinv * (p0_ref[...] + p1_ref[...] + t_ref[...]) + b_ref[...]
    h = jnp.maximum(h, 0.0)
    hw = jnp.dot(h, w_ref[...], preferred_element_type=jnp.float32)
    out_ref[...] = dinv * hw


def _tc_mid(p0, p1, t1, dinv, b1, w2):
    return pl.pallas_call(
        _mid_body,
        grid=(NP // _BLK,),
        in_specs=[
            pl.BlockSpec((_BLK, D), lambda i: (i, 0)),
            pl.BlockSpec((_BLK, D), lambda i: (i, 0)),
            pl.BlockSpec((_BLK, D), lambda i: (i, 0)),
            pl.BlockSpec((_BLK, 1), lambda i: (i, 0)),
            pl.BlockSpec((1, D), lambda i: (0, 0)),
            pl.BlockSpec((D, D), lambda i: (0, 0)),
        ],
        out_specs=pl.BlockSpec((_BLK, D), lambda i: (i, 0)),
        out_shape=jax.ShapeDtypeStruct((NP, D), jnp.float32),
    )(p0, p1, t1, dinv, b1, w2)


def _out_body(q0_ref, q1_ref, t_ref, dinv_ref, b_ref, out_ref):
    out_ref[...] = (dinv_ref[...] * (q0_ref[...] + q1_ref[...] + t_ref[...])
                    + b_ref[...])


def _tc_out(q0, q1, t2, dinv, b2):
    return pl.pallas_call(
        _out_body,
        grid=(NP // _BLK,),
        in_specs=[
            pl.BlockSpec((_BLK, D), lambda i: (i, 0)),
            pl.BlockSpec((_BLK, D), lambda i: (i, 0)),
            pl.BlockSpec((_BLK, D), lambda i: (i, 0)),
            pl.BlockSpec((_BLK, 1), lambda i: (i, 0)),
            pl.BlockSpec((1, D), lambda i: (0, 0)),
        ],
        out_specs=pl.BlockSpec((_BLK, D), lambda i: (i, 0)),
        out_shape=jax.ShapeDtypeStruct((NP, D), jnp.float32),
    )(q0, q1, t2, dinv, b2)


def kernel(x, edge_index, W1, b1, W2, b2):
    ei = edge_index.astype(jnp.int32)
    pad = EP * NW - E
    src = jnp.concatenate([ei[0], jnp.zeros((pad,), jnp.int32)]).reshape(NW, NCH, CH)
    dst = jnp.concatenate([ei[1], jnp.full((pad,), N, jnp.int32)]).reshape(NW, NCH, CH)
    x_p = jnp.pad(x, ((0, NP - N), (0, 0)))
    zero_rows = jnp.zeros((RPS, D), jnp.float32)
    zero_deg = jnp.zeros((RPS, DEGW), jnp.float32)
    ones_deg = jnp.ones((CH, DEGW), jnp.float32)

    degp = _sc_degree(dst, zero_deg, ones_deg)
    t1, dinv = _tc_prep(x_p, W1, degp[0], degp[1])
    p = _sc_scatter(t1, src, dst, zero_rows)
    t2 = _tc_mid(p[0], p[1], t1, dinv, b1.reshape(1, D), W2)
    q = _sc_scatter(t2, src, dst, zero_rows)
    out = _tc_out(q[0], q[1], t2, dinv, b2.reshape(1, D))
    return out[:N]


# speedup vs baseline: 13.6975x; 1.7251x over previous
import functools

import jax
import jax.numpy as jnp
from jax import lax
from jax.experimental import pallas as pl
from jax.experimental.pallas import tpu as pltpu
from jax.experimental.pallas import tpu_sc as plsc

N = 10000
NP = 10240
E = 320000
D = 128
W = 64
NC = 2
NS = 16
NW = NC * NS
CH = 128
EPW = 10240
NCH = EPW // CH
EPS = 20480
NCH2 = EPS // CH
RPS = NP // NS

_MESH = plsc.VectorSubcoreMesh(core_axis_name="c", subcore_axis_name="s",
                               num_cores=NC, num_subcores=NS)


_NSEM = 4


@functools.partial(
    pl.kernel,
    out_type=jax.ShapeDtypeStruct((NC, NP, D), jnp.float32),
    mesh=_MESH,
    scratch_types=[
        pltpu.VMEM((NCH, CH), jnp.int32),
        pltpu.VMEM((CH, D), jnp.float32),
        pltpu.VMEM_SHARED((NP, D), jnp.float32),
    ] + [pltpu.SemaphoreType.DMA] * _NSEM,
)
def _sc_degree(dst_hbm, zero_hbm, ones_hbm, out_hbm, idx_v, ones_v, acc, *sems):
    cid = lax.axis_index("c")
    sid = lax.axis_index("s")
    wid = sid * NC + cid
    pltpu.sync_copy(zero_hbm, acc.at[pl.ds(sid * RPS, RPS)])
    pltpu.sync_copy(ones_hbm, ones_v)
    pltpu.sync_copy(dst_hbm.at[wid], idx_v)
    plsc.subcore_barrier()

    def body(g, carry):
        for b in range(_NSEM):
            j = g * _NSEM + b

            @pl.when(g > 0)
            def _():
                pltpu.make_async_copy(ones_v, acc.at[pl.ds(0, CH)],
                                      sems[b]).wait()

            pltpu.async_copy(ones_v, acc.at[idx_v.at[j]], sems[b], add=True)
        return carry

    lax.fori_loop(0, NCH // _NSEM, body, 0)
    for b in range(_NSEM):
        pltpu.make_async_copy(ones_v, acc.at[pl.ds(0, CH)], sems[b]).wait()
    plsc.subcore_barrier()
    pltpu.sync_copy(acc.at[pl.ds(sid * RPS, RPS)],
                    out_hbm.at[cid, pl.ds(sid * RPS, RPS)])


_NBUF = 5


@functools.partial(
    pl.kernel,
    out_type=jax.ShapeDtypeStruct((NC, NP, W), jnp.float32),
    mesh=_MESH,
    compiler_params=pltpu.CompilerParams(use_tc_tiling_on_sc=False),
    scratch_types=[
        pltpu.VMEM((NCH2, CH), jnp.int32),
        pltpu.VMEM((NCH2, CH), jnp.int32),
        pltpu.VMEM((_NBUF, CH, W), jnp.float32),
        pltpu.VMEM_SHARED((NP, W), jnp.float32),
    ] + [pltpu.SemaphoreType.DMA] * (2 * _NBUF),
)
def _sc_scatter(table_hbm, src_hbm, dst_hbm, zero_hbm, out_hbm,
                idx_s, idx_d, rows_v, acc, *sems):
    gsem = sems[:_NBUF]
    ssem = sems[_NBUF:]
    cid = lax.axis_index("c")
    sid = lax.axis_index("s")
    pltpu.sync_copy(zero_hbm, acc.at[pl.ds(sid * RPS, RPS)])
    pltpu.sync_copy(src_hbm.at[sid], idx_s)
    pltpu.sync_copy(dst_hbm.at[sid], idx_d)
    plsc.subcore_barrier()

    def gather(j, b):
        pltpu.async_copy(table_hbm.at[cid].at[idx_s.at[j]], rows_v.at[b],
                         gsem[b])

    for b in range(_NBUF - 2):
        gather(b, b)

    def body(g, carry):
        for b in range(_NBUF):
            j = g * _NBUF + b
            bn = (b + 3) % _NBUF

            @pl.when(j >= 2)
            def _():
                pltpu.make_async_copy(rows_v.at[bn], acc.at[pl.ds(0, CH)],
                                      ssem[bn]).wait()

            @pl.when(j < NCH2 - 3)
            def _():
                gather(j + 3, bn)

            pltpu.make_async_copy(zero_hbm.at[pl.ds(0, CH)], rows_v.at[b],
                                  gsem[b]).wait()
            pltpu.async_copy(rows_v.at[b], acc.at[idx_d.at[j]], ssem[b],
                             add=True)
        return carry

    lax.fori_loop(0, NCH2 // _NBUF, body, 0)
    for j in (NCH2 - 2, NCH2 - 1):
        b = j % _NBUF
        pltpu.make_async_copy(rows_v.at[b], acc.at[pl.ds(0, CH)],
                              ssem[b]).wait()
    plsc.subcore_barrier()
    pltpu.sync_copy(acc.at[pl.ds(sid * RPS, RPS)],
                    out_hbm.at[cid, pl.ds(sid * RPS, RPS)])


_BLK = 1280


def _prep_body(x_ref, w_ref, d0_ref, d1_ref, t_ref, dinv_ref):
    dp = d0_ref[...] + d1_ref[...]
    deg = 1.0 + jnp.sum(dp, axis=1, keepdims=True) * (1.0 / D)
    dinv = lax.rsqrt(deg)
    xw = jnp.dot(x_ref[...], w_ref[...], preferred_element_type=jnp.float32)
    t_ref[...] = dinv * xw
    dinv_ref[...] = dinv


def _tc_prep(x, w1, d0, d1):
    return pl.pallas_call(
        _prep_body,
        grid=(NP // _BLK,),
        in_specs=[
            pl.BlockSpec((_BLK, D), lambda i: (i, 0)),
            pl.BlockSpec((D, D), lambda i: (0, 0)),
            pl.BlockSpec((_BLK, D), lambda i: (i, 0)),
            pl.BlockSpec((_BLK, D), lambda i: (i, 0)),
        ],
        out_specs=[
            pl.BlockSpec((_BLK, D), lambda i: (i, 0)),
            pl.BlockSpec((_BLK, 1), lambda i: (i, 0)),
        ],
        out_shape=[
            jax.ShapeDtypeStruct((NP, D), jnp.float32),
            jax.ShapeDtypeStruct((NP, 1), jnp.float32),
        ],
    )(x, w1, d0, d1)


def _mid_body(pa_ref, pb_ref, t_ref, dinv_ref, b_ref, w_ref, out_ref):
    s = jnp.concatenate([pa_ref[0], pb_ref[0]], axis=-1)
    dinv = dinv_ref[...]
    h = dinv * (s + t_ref[...]) + b_ref[...]
    h = jnp.maximum(h, 0.0)
    hw = jnp.dot(h, w_ref[...], preferred_element_type=jnp.float32)
    out_ref[...] = dinv * hw


def _tc_mid(p, t1, dinv, b1, w2):
    return pl.pallas_call(
        _mid_body,
        grid=(NP // _BLK,),
        in_specs=[
            pl.BlockSpec((1, _BLK, W), lambda i: (0, i, 0)),
            pl.BlockSpec((1, _BLK, W), lambda i: (1, i, 0)),
            pl.BlockSpec((_BLK, D), lambda i: (i, 0)),
            pl.BlockSpec((_BLK, 1), lambda i: (i, 0)),
            pl.BlockSpec((1, D), lambda i: (0, 0)),
            pl.BlockSpec((D, D), lambda i: (0, 0)),
        ],
        out_specs=pl.BlockSpec((_BLK, D), lambda i: (i, 0)),
        out_shape=jax.ShapeDtypeStruct((NP, D), jnp.float32),
    )(p, p, t1, dinv, b1, w2)


def _out_body(qa_ref, qb_ref, t_ref, dinv_ref, b_ref, out_ref):
    s = jnp.concatenate([qa_ref[0], qb_ref[0]], axis=-1)
    out_ref[...] = dinv_ref[...] * (s + t_ref[...]) + b_ref[...]


def _tc_out(q, t2, dinv, b2):
    return pl.pallas_call(
        _out_body,
        grid=(NP // _BLK,),
        in_specs=[
            pl.BlockSpec((1, _BLK, W), lambda i: (0, i, 0)),
            pl.BlockSpec((1, _BLK, W), lambda i: (1, i, 0)),
            pl.BlockSpec((_BLK, D), lambda i: (i, 0)),
            pl.BlockSpec((_BLK, 1), lambda i: (i, 0)),
            pl.BlockSpec((1, D), lambda i: (0, 0)),
        ],
        out_specs=pl.BlockSpec((_BLK, D), lambda i: (i, 0)),
        out_shape=jax.ShapeDtypeStruct((NP, D), jnp.float32),
    )(q, q, t2, dinv, b2)


def kernel(x, edge_index, W1, b1, W2, b2):
    ei = edge_index.astype(jnp.int32)
    padw = EPW * NW - E
    dstw = jnp.concatenate([ei[1], jnp.full((padw,), N, jnp.int32)]
                           ).reshape(NW, NCH, CH)
    pads = EPS * NS - E
    srcs = jnp.concatenate([ei[0], jnp.zeros((pads,), jnp.int32)]
                           ).reshape(NS, NCH2, CH)
    dsts = jnp.concatenate([ei[1], jnp.full((pads,), N, jnp.int32)]
                           ).reshape(NS, NCH2, CH)
    x_p = jnp.pad(x, ((0, NP - N), (0, 0)))
    zero_w = jnp.zeros((RPS, W), jnp.float32)
    zero_deg = jnp.zeros((RPS, D), jnp.float32)
    ones_deg = jnp.ones((CH, D), jnp.float32)

    degp = _sc_degree(dstw, zero_deg, ones_deg)
    t1, dinv = _tc_prep(x_p, W1, degp[0], degp[1])
    t1s = jnp.stack([t1[:, :W], t1[:, W:]])
    p = _sc_scatter(t1s, srcs, dsts, zero_w)
    t2 = _tc_mid(p, t1, dinv, b1.reshape(1, D), W2)
    t2s = jnp.stack([t2[:, :W], t2[:, W:]])
    q = _sc_scatter(t2s, srcs, dsts, zero_w)
    out = _tc_out(q, t2, dinv, b2.reshape(1, D))
    return out[:N]
